# baseline (speedup 1.0000x reference)
import jax
import jax.numpy as jnp
from jax import lax
from jax.experimental import pallas as pl
from jax.experimental.pallas import tpu as pltpu
from jax.experimental.pallas import tpu_sc as plsc

_N = 10000
_E2 = 640000
_T = 64
_A = 64
_H = 128
_W = 48
_NW = 32
_NP = 10240
_CHUNK = 512
_NCHUNK = 40
_EP = _NW * _NCHUNK * _CHUNK
_GCHUNK = 80
_STRIPE = _NP // 16
_NGC = _STRIPE // _GCHUNK
_BLK = 512


def _table_body(te_ref, ae_ref, w0_ref, b0_ref, g0_ref, be0_ref,
                outa_ref, outb_ref, outc_ref):
    p = jnp.dot(te_ref[...], w0_ref[0:_T, :], preferred_element_type=jnp.float32)
    q = jnp.dot(ae_ref[...], w0_ref[_T:2 * _T, :], preferred_element_type=jnp.float32)
    h = p[:, None, :] + q[None, :, :] + b0_ref[...][None, :, :]
    h = jnp.maximum(h, 0.0).reshape(_T * _A, _H)
    mu = jnp.mean(h, axis=-1, keepdims=True)
    var = jnp.mean((h - mu) ** 2, axis=-1, keepdims=True)
    ln = (h - mu) / jnp.sqrt(var + 1e-5) * g0_ref[...] + be0_ref[...]
    outa_ref[...] = ln[:, 0:_W]
    outb_ref[...] = ln[:, _W:2 * _W]
    ones = jnp.ones((_T * _A, 1), jnp.float32)
    zeros = jnp.zeros((_T * _A, 3 * _W - _H - 1), jnp.float32)
    outc_ref[...] = jnp.concatenate([ln[:, 2 * _W:_H], ones, zeros], axis=-1)


def _build_table(type_embed, agent_embed, W0, b0, g0, be0):
    return pl.pallas_call(
        _table_body,
        out_shape=[jax.ShapeDtypeStruct((_T * _A, _W), jnp.float32)] * 3,
    )(type_embed, agent_embed, W0, b0.reshape(1, _H), g0.reshape(1, _H),
      be0.reshape(1, _H))


def _make_agg_body():
    def body(comb_ref, table_ref, src_ref, dst_ref, zeros_ref,
             xh_ref, aggp_ref,
             nidx, nrows, s0, s1, d0, d1, r0, r1,
             nsem, i0, i1, g0, g1, x_sh, agg_sh):
        c = lax.axis_index("c")
        s = lax.axis_index("s")
        w = s * 2 + c
        stages = ((s0, d0, r0, i0, g0), (s1, d1, r1, i1, g1))

        def fire_idx(k, st):
            pltpu.async_copy(src_ref.at[w, k], st[0], st[3])
            pltpu.async_copy(dst_ref.at[w, k], st[1], st[3])

        def fire_gather(st):
            pltpu.make_async_copy(src_ref.at[w, 0], st[0], st[3]).wait()
            pltpu.make_async_copy(dst_ref.at[w, 0], st[1], st[3]).wait()
            pltpu.async_copy(x_sh.at[st[0]], st[2], st[4])

        def wait_gather(st):
            pltpu.make_async_copy(x_sh.at[st[0]], st[2], st[4]).wait()

        for p in range(3):
            pltpu.sync_copy(zeros_ref, agg_sh.at[pl.ds(s * _STRIPE, _STRIPE)])
            for j in range(_NGC):
                pltpu.sync_copy(comb_ref.at[s, j], nidx)
                pltpu.async_copy(table_ref.at[p].at[nidx], nrows, nsem).wait()
                pltpu.sync_copy(
                    nrows, x_sh.at[pl.ds(s * _STRIPE + j * _GCHUNK, _GCHUNK)])

            @pl.when(c == 0)
            def _():
                pltpu.sync_copy(x_sh.at[pl.ds(s * _STRIPE, _STRIPE)],
                                xh_ref.at[p].at[pl.ds(s * _STRIPE, _STRIPE)])

            plsc.subcore_barrier()

            fire_idx(0, stages[0])
            fire_idx(1, stages[1])
            fire_gather(stages[0])

            def group(kk, carry):
                for b in range(2):
                    k = kk * 2 + b

                    @pl.when(k + 1 < _NCHUNK)
                    def _():
                        fire_gather(stages[1 - b])

                    wait_gather(stages[b])
                    pltpu.sync_copy(stages[b][2], agg_sh.at[stages[b][1]],
                                    add=True)

                    @pl.when(k + 2 < _NCHUNK)
                    def _():
                        fire_idx(k + 2, stages[b])
                return carry

            lax.fori_loop(0, _NCHUNK // 2, group, 0)
            plsc.subcore_barrier()

            pltpu.sync_copy(
                agg_sh.at[pl.ds(s * _STRIPE, _STRIPE)],
                aggp_ref.at[p].at[c].at[pl.ds(s * _STRIPE, _STRIPE)])
            plsc.subcore_barrier()

    return body


def _edge_aggregate(comb_s, tables, src_idx, dst_idx, zeros_stripe):
    mesh = plsc.VectorSubcoreMesh(core_axis_name="c", subcore_axis_name="s")
    idx = pltpu.VMEM((_CHUNK,), jnp.int32)
    rows = pltpu.VMEM((_CHUNK, _W), jnp.float32)
    sem = pltpu.SemaphoreType.DMA
    return pl.kernel(
        _make_agg_body(),
        out_type=[jax.ShapeDtypeStruct((3, _NP, _W), jnp.float32),
                  jax.ShapeDtypeStruct((3, 2, _NP, _W), jnp.float32)],
        mesh=mesh,
        compiler_params=pltpu.CompilerParams(use_tc_tiling_on_sc=False),
        scratch_types=[
            pltpu.VMEM((_GCHUNK,), jnp.int32),
            pltpu.VMEM((_GCHUNK, _W), jnp.float32),
            idx, idx, idx, idx, rows, rows,
            sem, sem, sem, sem, sem,
            pltpu.VMEM_SHARED((_NP, _W), jnp.float32),
            pltpu.VMEM_SHARED((_NP, _W), jnp.float32),
        ],
    )(comb_s, tables, src_idx, dst_idx, zeros_stripe)


def _final_body(xa_ref, xb_ref, xc_ref, pa_ref, pb_ref, pc_ref,
                w1_ref, b1_ref, g1_ref, be1_ref, wp_ref, bp_ref,
                out_ref, acc_ref):
    i = pl.program_id(0)

    @pl.when(i == 0)
    def _():
        acc_ref[...] = jnp.zeros_like(acc_ref)

    sa = pa_ref[0] + pa_ref[1]
    sb = pb_ref[0] + pb_ref[1]
    sc = pc_ref[0] + pc_ref[1]
    w3 = _H - 2 * _W
    denom = jnp.maximum(sc[:, w3:w3 + 1], 1.0)
    h = (jnp.dot(xa_ref[...], w1_ref[0:_W, :],
                 preferred_element_type=jnp.float32)
         + jnp.dot(xb_ref[...], w1_ref[_W:2 * _W, :],
                   preferred_element_type=jnp.float32)
         + jnp.dot(xc_ref[:, 0:w3], w1_ref[2 * _W:_H, :],
                   preferred_element_type=jnp.float32)
         + jnp.dot(sa / denom, w1_ref[_H:_H + _W, :],
                   preferred_element_type=jnp.float32)
         + jnp.dot(sb / denom, w1_ref[_H + _W:_H + 2 * _W, :],
                   preferred_element_type=jnp.float32)
         + jnp.dot(sc[:, 0:w3] / denom, w1_ref[_H + 2 * _W:2 * _H, :],
                   preferred_element_type=jnp.float32)
         + b1_ref[...])
    h = jnp.maximum(h, 0.0)
    mu = jnp.mean(h, axis=-1, keepdims=True)
    var = jnp.mean((h - mu) ** 2, axis=-1, keepdims=True)
    ln = (h - mu) / jnp.sqrt(var + 1e-5) * g1_ref[...] + be1_ref[...]
    rows = lax.broadcasted_iota(jnp.int32, (_BLK, 1), 0) + i * _BLK
    ln = jnp.where(rows < _N, ln, 0.0)
    acc_ref[...] += jnp.sum(ln, axis=0, keepdims=True)

    @pl.when(i == pl.num_programs(0) - 1)
    def _():
        ge = acc_ref[...] / float(_N)
        out_ref[...] = (jnp.dot(ge, wp_ref[...],
                                preferred_element_type=jnp.float32)
                        + bp_ref[...])


def _finalize(xs, ps, W1, b1, g1, be1, Wp, bp):
    nblk = _NP // _BLK
    return pl.pallas_call(
        _final_body,
        grid=(nblk,),
        in_specs=[pl.BlockSpec((_BLK, _W), lambda i: (i, 0))] * 3
        + [pl.BlockSpec((2, _BLK, _W), lambda i: (0, i, 0))] * 3
        + [
            pl.BlockSpec((2 * _H, _H), lambda i: (0, 0)),
            pl.BlockSpec((1, _H), lambda i: (0, 0)),
            pl.BlockSpec((1, _H), lambda i: (0, 0)),
            pl.BlockSpec((1, _H), lambda i: (0, 0)),
            pl.BlockSpec((_H, _H), lambda i: (0, 0)),
            pl.BlockSpec((1, _H), lambda i: (0, 0)),
        ],
        out_specs=pl.BlockSpec((1, _H), lambda i: (0, 0)),
        out_shape=jax.ShapeDtypeStruct((1, _H), jnp.float32),
        scratch_shapes=[pltpu.VMEM((1, _H), jnp.float32)],
    )(*xs, *ps, W1, b1.reshape(1, _H), g1.reshape(1, _H),
      be1.reshape(1, _H), Wp, bp.reshape(1, _H))


def kernel(type_idx, agent_idx, edge_index, type_embed, agent_embed,
           W0, b0, g0, be0, W1, b1, g1, be1, Wp, bp):
    type_idx = type_idx.astype(jnp.int32)
    agent_idx = agent_idx.astype(jnp.int32)
    edge_index = edge_index.astype(jnp.int32)

    comb = type_idx * _A + agent_idx
    comb_s = jnp.concatenate(
        [comb, jnp.zeros((_NP - _N,), jnp.int32)]).reshape(16, _NGC, _GCHUNK)
    pad = jnp.full((_EP - _E2,), _N, jnp.int32)
    src = jnp.concatenate([edge_index[0], edge_index[1], pad]).reshape(
        _NW, _NCHUNK, _CHUNK)
    dst = jnp.concatenate([edge_index[1], edge_index[0], pad]).reshape(
        _NW, _NCHUNK, _CHUNK)
    z = jnp.zeros((_STRIPE, _W), jnp.float32)

    tables = jnp.stack(_build_table(type_embed, agent_embed, W0, b0, g0, be0))
    xh, aggp = _edge_aggregate(comb_s, tables, src, dst, z)
    xs = [xh[0], xh[1], xh[2]]
    ps = [aggp[0], aggp[1], aggp[2]]
    out = _finalize(xs, ps, W1, b1, g1, be1, Wp, bp)
    return out.reshape(_H)


# speedup vs baseline: 1.1012x; 1.1012x over previous
import jax
import jax.numpy as jnp
from jax import lax
from jax.experimental import pallas as pl
from jax.experimental.pallas import tpu as pltpu
from jax.experimental.pallas import tpu_sc as plsc

_N = 10000
_E2 = 640000
_T = 64
_A = 64
_H = 128
_W = 48
_NW = 32
_NP = 10240
_CHUNK = 512
_NCHUNK = 40
_EP = _NW * _NCHUNK * _CHUNK
_GCHUNK = 320
_STRIPE = _NP // 16
_NGC = _STRIPE // _GCHUNK
_BLK = 512


def _table_body(te_ref, ae_ref, w0_ref, b0_ref, g0_ref, be0_ref,
                outa_ref, outb_ref, outc_ref):
    p = jnp.dot(te_ref[...], w0_ref[0:_T, :], preferred_element_type=jnp.float32)
    q = jnp.dot(ae_ref[...], w0_ref[_T:2 * _T, :], preferred_element_type=jnp.float32)
    h = p[:, None, :] + q[None, :, :] + b0_ref[...][None, :, :]
    h = jnp.maximum(h, 0.0).reshape(_T * _A, _H)
    mu = jnp.mean(h, axis=-1, keepdims=True)
    var = jnp.mean((h - mu) ** 2, axis=-1, keepdims=True)
    ln = (h - mu) / jnp.sqrt(var + 1e-5) * g0_ref[...] + be0_ref[...]
    outa_ref[...] = ln[:, 0:_W]
    outb_ref[...] = ln[:, _W:2 * _W]
    ones = jnp.ones((_T * _A, 1), jnp.float32)
    zeros = jnp.zeros((_T * _A, 3 * _W - _H - 1), jnp.float32)
    outc_ref[...] = jnp.concatenate([ln[:, 2 * _W:_H], ones, zeros], axis=-1)


def _build_table(type_embed, agent_embed, W0, b0, g0, be0):
    return pl.pallas_call(
        _table_body,
        out_shape=[jax.ShapeDtypeStruct((_T * _A, _W), jnp.float32)] * 3,
    )(type_embed, agent_embed, W0, b0.reshape(1, _H), g0.reshape(1, _H),
      be0.reshape(1, _H))


def _make_agg_body():
    def body(comb_ref, table_ref, src_ref, dst_ref, zeros_ref,
             xh_ref, aggp_ref,
             nidx, nrows, s0, s1, d0, d1, r0, r1,
             nsem, i0, i1, g0, g1, x_sh, agg_sh):
        c = lax.axis_index("c")
        s = lax.axis_index("s")
        w = s * 2 + c
        stages = ((s0, d0, r0, i0, g0), (s1, d1, r1, i1, g1))

        pltpu.sync_copy(zeros_ref, agg_sh.at[pl.ds(s * _STRIPE, _STRIPE)])
        for j in range(_NGC):
            pltpu.sync_copy(comb_ref.at[s, j], nidx)
            pltpu.async_copy(table_ref.at[nidx], nrows, nsem).wait()
            pltpu.sync_copy(
                nrows, x_sh.at[pl.ds(s * _STRIPE + j * _GCHUNK, _GCHUNK)])

        @pl.when(c == 0)
        def _():
            pltpu.sync_copy(x_sh.at[pl.ds(s * _STRIPE, _STRIPE)],
                            xh_ref.at[pl.ds(s * _STRIPE, _STRIPE)])

        def fire_idx(k, st):
            pltpu.async_copy(src_ref.at[w, k], st[0], st[3])
            pltpu.async_copy(dst_ref.at[w, k], st[1], st[3])

        def fire_gather(st):
            pltpu.make_async_copy(src_ref.at[w, 0], st[0], st[3]).wait()
            pltpu.make_async_copy(dst_ref.at[w, 0], st[1], st[3]).wait()
            pltpu.async_copy(x_sh.at[st[0]], st[2], st[4])

        def wait_gather(st):
            pltpu.make_async_copy(x_sh.at[st[0]], st[2], st[4]).wait()

        plsc.subcore_barrier()

        fire_idx(0, stages[0])
        fire_idx(1, stages[1])
        fire_gather(stages[0])

        def group(kk, carry):
            for b in range(2):
                k = kk * 2 + b

                @pl.when(k + 1 < _NCHUNK)
                def _():
                    fire_gather(stages[1 - b])

                wait_gather(stages[b])
                pltpu.sync_copy(stages[b][2], agg_sh.at[stages[b][1]],
                                add=True)

                @pl.when(k + 2 < _NCHUNK)
                def _():
                    fire_idx(k + 2, stages[b])
            return carry

        lax.fori_loop(0, _NCHUNK // 2, group, 0)
        plsc.subcore_barrier()

        pltpu.sync_copy(agg_sh.at[pl.ds(s * _STRIPE, _STRIPE)],
                        aggp_ref.at[c].at[pl.ds(s * _STRIPE, _STRIPE)])

    return body


def _edge_aggregate(comb_s, table_h, src_idx, dst_idx, zeros_stripe):
    mesh = plsc.VectorSubcoreMesh(core_axis_name="c", subcore_axis_name="s")
    idx = pltpu.VMEM((_CHUNK,), jnp.int32)
    rows = pltpu.VMEM((_CHUNK, _W), jnp.float32)
    sem = pltpu.SemaphoreType.DMA
    return pl.kernel(
        _make_agg_body(),
        out_type=[jax.ShapeDtypeStruct((_NP, _W), jnp.float32),
                  jax.ShapeDtypeStruct((2, _NP, _W), jnp.float32)],
        mesh=mesh,
        compiler_params=pltpu.CompilerParams(use_tc_tiling_on_sc=False),
        scratch_types=[
            pltpu.VMEM((_GCHUNK,), jnp.int32),
            pltpu.VMEM((_GCHUNK, _W), jnp.float32),
            idx, idx, idx, idx, rows, rows,
            sem, sem, sem, sem, sem,
            pltpu.VMEM_SHARED((_NP, _W), jnp.float32),
            pltpu.VMEM_SHARED((_NP, _W), jnp.float32),
        ],
    )(comb_s, table_h, src_idx, dst_idx, zeros_stripe)


def _final_body(xa_ref, xb_ref, xc_ref, pa_ref, pb_ref, pc_ref,
                w1_ref, b1_ref, g1_ref, be1_ref, wp_ref, bp_ref,
                out_ref, acc_ref):
    i = pl.program_id(0)

    @pl.when(i == 0)
    def _():
        acc_ref[...] = jnp.zeros_like(acc_ref)

    sa = pa_ref[0] + pa_ref[1]
    sb = pb_ref[0] + pb_ref[1]
    sc = pc_ref[0] + pc_ref[1]
    w3 = _H - 2 * _W
    denom = jnp.maximum(sc[:, w3:w3 + 1], 1.0)
    h = (jnp.dot(xa_ref[...], w1_ref[0:_W, :],
                 preferred_element_type=jnp.float32)
         + jnp.dot(xb_ref[...], w1_ref[_W:2 * _W, :],
                   preferred_element_type=jnp.float32)
         + jnp.dot(xc_ref[:, 0:w3], w1_ref[2 * _W:_H, :],
                   preferred_element_type=jnp.float32)
         + jnp.dot(sa / denom, w1_ref[_H:_H + _W, :],
                   preferred_element_type=jnp.float32)
         + jnp.dot(sb / denom, w1_ref[_H + _W:_H + 2 * _W, :],
                   preferred_element_type=jnp.float32)
         + jnp.dot(sc[:, 0:w3] / denom, w1_ref[_H + 2 * _W:2 * _H, :],
                   preferred_element_type=jnp.float32)
         + b1_ref[...])
    h = jnp.maximum(h, 0.0)
    mu = jnp.mean(h, axis=-1, keepdims=True)
    var = jnp.mean((h - mu) ** 2, axis=-1, keepdims=True)
    ln = (h - mu) / jnp.sqrt(var + 1e-5) * g1_ref[...] + be1_ref[...]
    rows = lax.broadcasted_iota(jnp.int32, (_BLK, 1), 0) + i * _BLK
    ln = jnp.where(rows < _N, ln, 0.0)
    acc_ref[...] += jnp.sum(ln, axis=0, keepdims=True)

    @pl.when(i == pl.num_programs(0) - 1)
    def _():
        ge = acc_ref[...] / float(_N)
        out_ref[...] = (jnp.dot(ge, wp_ref[...],
                                preferred_element_type=jnp.float32)
                        + bp_ref[...])


def _finalize(xs, ps, W1, b1, g1, be1, Wp, bp):
    nblk = _NP // _BLK
    return pl.pallas_call(
        _final_body,
        grid=(nblk,),
        in_specs=[pl.BlockSpec((_BLK, _W), lambda i: (i, 0))] * 3
        + [pl.BlockSpec((2, _BLK, _W), lambda i: (0, i, 0))] * 3
        + [
            pl.BlockSpec((2 * _H, _H), lambda i: (0, 0)),
            pl.BlockSpec((1, _H), lambda i: (0, 0)),
            pl.BlockSpec((1, _H), lambda i: (0, 0)),
            pl.BlockSpec((1, _H), lambda i: (0, 0)),
            pl.BlockSpec((_H, _H), lambda i: (0, 0)),
            pl.BlockSpec((1, _H), lambda i: (0, 0)),
        ],
        out_specs=pl.BlockSpec((1, _H), lambda i: (0, 0)),
        out_shape=jax.ShapeDtypeStruct((1, _H), jnp.float32),
        scratch_shapes=[pltpu.VMEM((1, _H), jnp.float32)],
    )(*xs, *ps, W1, b1.reshape(1, _H), g1.reshape(1, _H),
      be1.reshape(1, _H), Wp, bp.reshape(1, _H))


def kernel(type_idx, agent_idx, edge_index, type_embed, agent_embed,
           W0, b0, g0, be0, W1, b1, g1, be1, Wp, bp):
    type_idx = type_idx.astype(jnp.int32)
    agent_idx = agent_idx.astype(jnp.int32)
    edge_index = edge_index.astype(jnp.int32)

    comb = type_idx * _A + agent_idx
    comb_s = jnp.concatenate(
        [comb, jnp.zeros((_NP - _N,), jnp.int32)]).reshape(16, _NGC, _GCHUNK)
    pad = jnp.full((_EP - _E2,), _N, jnp.int32)
    src = jnp.concatenate([edge_index[0], edge_index[1], pad]).reshape(
        _NW, _NCHUNK, _CHUNK)
    dst = jnp.concatenate([edge_index[1], edge_index[0], pad]).reshape(
        _NW, _NCHUNK, _CHUNK)
    z = jnp.zeros((_STRIPE, _W), jnp.float32)

    tables = _build_table(type_embed, agent_embed, W0, b0, g0, be0)
    xs, ps = [], []
    for t in tables:
        xh, aggp = _edge_aggregate(comb_s, t, src, dst, z)
        xs.append(xh)
        ps.append(aggp)
    out = _finalize(xs, ps, W1, b1, g1, be1, Wp, bp)
    return out.reshape(_H)
